# baseline (speedup 1.0000x reference)
import jax
import jax.numpy as jnp
from jax import lax
from jax.experimental import pallas as pl
from jax.experimental.pallas import tpu as pltpu
from jax.experimental.pallas import tpu_sc as plsc

B = 1024
N_PATCHES = 128
D_MODEL = 768
ROWS = B * N_PATCHES
BB = 32

NC = 2
NS = 16
NW = NC * NS
B_SC = 416
B_PER_W = B_SC // NW
HALF_N = N_PATCHES // 2
NVEC = D_MODEL // 16


def _tc_body(rp_ref, w_ref, s_ref, out_ref):
    wv = (w_ref[...] * s_ref[0, 0]).reshape(1, 1, D_MODEL)
    out_ref[...] = rp_ref[...][:, :, None] * wv


def _tc_kernel(relative_positions, W, scale):
    nb = relative_positions.shape[0]
    w2 = W.reshape(1, D_MODEL)
    s2 = scale.reshape(1, 1)
    grid = (nb // BB,)
    out = pl.pallas_call(
        _tc_body,
        grid=grid,
        in_specs=[
            pl.BlockSpec((BB, N_PATCHES), lambda i: (i, 0)),
            pl.BlockSpec((1, D_MODEL), lambda i: (0, 0)),
            pl.BlockSpec((1, 1), lambda i: (0, 0)),
        ],
        out_specs=pl.BlockSpec((BB, N_PATCHES, D_MODEL), lambda i: (i, 0, 0)),
        out_shape=jax.ShapeDtypeStruct((nb, N_PATCHES, D_MODEL), jnp.float32),
    )(relative_positions, w2, s2)
    return out


def _sc_body(rp_hbm, w_hbm, s_hbm, out_hbm, w_v, s_v, rp_v, buf, sems):
    cid = lax.axis_index("c")
    sid = lax.axis_index("s")
    wid = sid * NC + cid
    base = wid * B_PER_W

    pltpu.sync_copy(w_hbm, w_v)
    pltpu.sync_copy(s_hbm, s_v)
    pltpu.sync_copy(
        rp_hbm.at[pl.ds(base * N_PATCHES, B_PER_W * N_PATCHES)], rp_v
    )

    sv = s_v[...][0]

    def scale_w(j, carry):
        w_v[pl.ds(j * 16, 16)] = w_v[pl.ds(j * 16, 16)] * sv
        return carry

    lax.fori_loop(0, NVEC, scale_w, 0)

    def batch_body(bi, carry):
        for h in range(2):
            @pl.when(bi >= 1)
            def _reclaim():
                pltpu.make_async_copy(
                    buf.at[h], out_hbm.at[pl.ds(0, HALF_N), :], sems.at[h]
                ).wait()

            def row16_body(n16, c2):
                rvec = rp_v[pl.ds(bi * N_PATCHES + h * HALF_N + n16 * 16, 16)]
                for jg in range(NVEC // 8):
                    wregs = [w_v[pl.ds((jg * 8 + j) * 16, 16)] for j in range(8)]
                    for k in range(16):
                        rv = rvec[k]
                        n = n16 * 16 + k
                        for j in range(8):
                            buf[h, n, pl.ds((jg * 8 + j) * 16, 16)] = wregs[j] * rv
                return c2

            lax.fori_loop(0, HALF_N // 16, row16_body, 0)

            row0 = (base + bi) * N_PATCHES + h * HALF_N
            pltpu.make_async_copy(
                buf.at[h], out_hbm.at[pl.ds(row0, HALF_N), :], sems.at[h]
            ).start()
        return carry

    lax.fori_loop(0, B_PER_W, batch_body, 0)

    for h in range(2):
        pltpu.make_async_copy(
            buf.at[h], out_hbm.at[pl.ds(0, HALF_N), :], sems.at[h]
        ).wait()


def _sc_kernel(relative_positions, W, scale):
    import functools

    mesh = plsc.VectorSubcoreMesh(core_axis_name="c", subcore_axis_name="s")
    run = pl.kernel(
        _sc_body,
        mesh=mesh,
        out_type=jax.ShapeDtypeStruct((B_SC * N_PATCHES, D_MODEL), jnp.float32),
        scratch_types=[
            pltpu.VMEM((D_MODEL,), jnp.float32),
            pltpu.VMEM((16,), jnp.float32),
            pltpu.VMEM((B_PER_W * N_PATCHES,), jnp.float32),
            pltpu.VMEM((2, HALF_N, D_MODEL), jnp.float32),
            pltpu.SemaphoreType.DMA((2,)),
        ],
    )
    out = run(
        relative_positions.reshape(-1), W.reshape(D_MODEL),
        jnp.broadcast_to(scale, (16,)),
    )
    return out.reshape(B_SC, N_PATCHES, D_MODEL)


def kernel(n_patches, relative_positions, W, scale):
    sc_out = _sc_kernel(relative_positions[:B_SC], W, scale)
    tc_out = _tc_kernel(relative_positions[B_SC:], W, scale)
    return jnp.concatenate([sc_out, tc_out], axis=0)


# speedup vs baseline: 3.2061x; 3.2061x over previous
import jax
import jax.numpy as jnp
from jax import lax
from jax.experimental import pallas as pl
from jax.experimental.pallas import tpu as pltpu

B = 1024
N_PATCHES = 128
D_MODEL = 768
BB = 16
NBUF = 4
NSTEP = B // BB


def _body(rp_ref, w_ref, s_ref, out_hbm, buf, sems):
    i = pl.program_id(0)
    slot = lax.rem(i, NBUF)
    wv = (w_ref[...] * s_ref[0, 0]).reshape(1, 1, D_MODEL)

    @pl.when(i >= NBUF)
    def _reclaim():
        pltpu.make_async_copy(
            buf.at[slot], out_hbm.at[pl.ds(i * BB, BB), :, :], sems.at[slot]
        ).wait()

    buf[slot] = rp_ref[...][:, :, None] * wv
    pltpu.make_async_copy(
        buf.at[slot], out_hbm.at[pl.ds(i * BB, BB), :, :], sems.at[slot]
    ).start()

    @pl.when(i == NSTEP - 1)
    def _drain():
        for k in range(NBUF):
            pltpu.make_async_copy(
                buf.at[k], out_hbm.at[pl.ds(0, BB), :, :], sems.at[k]
            ).wait()


def kernel(n_patches, relative_positions, W, scale):
    w2 = W.reshape(1, D_MODEL)
    s2 = scale.reshape(1, 1)
    out = pl.pallas_call(
        _body,
        grid=(NSTEP,),
        in_specs=[
            pl.BlockSpec((BB, N_PATCHES), lambda i: (i, 0)),
            pl.BlockSpec((1, D_MODEL), lambda i: (0, 0)),
            pl.BlockSpec((1, 1), lambda i: (0, 0)),
        ],
        out_specs=pl.BlockSpec(memory_space=pl.ANY),
        out_shape=jax.ShapeDtypeStruct((B, N_PATCHES, D_MODEL), jnp.float32),
        scratch_shapes=[
            pltpu.VMEM((NBUF, BB, N_PATCHES, D_MODEL), jnp.float32),
            pltpu.SemaphoreType.DMA((NBUF,)),
        ],
    )(relative_positions, w2, s2)
    return out
